# Initial kernel scaffold; baseline (speedup 1.0000x reference)
import functools

import jax
import jax.numpy as jnp
from jax import lax
from jax.experimental import pallas as pl
from jax.experimental.pallas import tpu as pltpu
from jax.experimental.pallas import tpu_sc as plsc

N_NODES = 100000
N_GRAPHS = 100
ROWS_PER_GRAPH = 1000
FDIM = 512
LANES = 16
VECS = FDIM // LANES
CHUNK = 100
N_CHUNKS = ROWS_PER_GRAPH // CHUNK
NW = 32
GRAPHS_PER_W = 4


def _pool_body(feat_hbm, nn_hbm, out_hbm, buf0, buf1, nn_v, out_buf,
               sem0, sem1):
    wid = lax.axis_index("c") * 16 + lax.axis_index("s")
    pltpu.sync_copy(nn_hbm, nn_v)
    bufs = (buf0, buf1)
    sems = (sem0, sem1)

    for k in range(GRAPHS_PER_W):
        g = k * NW + wid

        @pl.when(g < N_GRAPHS)
        def _():
            row0 = g * ROWS_PER_GRAPH
            pltpu.async_copy(
                feat_hbm.at[pl.ds(row0, CHUNK), :], bufs[0], sems[0]).wait()
            accs = tuple(jnp.zeros((LANES,), jnp.float32)
                         for _ in range(VECS))
            for i in range(N_CHUNKS):
                cur = bufs[i % 2]
                if i + 1 < N_CHUNKS:
                    nxt = pltpu.async_copy(
                        feat_hbm.at[pl.ds(row0 + (i + 1) * CHUNK, CHUNK), :],
                        bufs[(i + 1) % 2], sems[(i + 1) % 2])

                def row_body(r, accs):
                    return tuple(
                        accs[j] + cur[r, pl.ds(j * LANES, LANES)]
                        for j in range(VECS))

                accs = lax.fori_loop(0, CHUNK, row_body, accs)
                if i + 1 < N_CHUNKS:
                    nxt.wait()

            n_f = jnp.maximum(nn_v[g].astype(jnp.float32), 1.0)
            scale = 1.0 / jnp.full((LANES,), n_f, jnp.float32)
            for j in range(VECS):
                out_buf[0, pl.ds(j * LANES, LANES)] = accs[j] * scale
            pltpu.sync_copy(out_buf, out_hbm.at[pl.ds(g, 1), :])


@jax.jit
def _pool(feat2d, n_nodes):
    mesh = plsc.VectorSubcoreMesh(core_axis_name="c", subcore_axis_name="s")
    f = functools.partial(
        pl.kernel,
        mesh=mesh,
        out_type=jax.ShapeDtypeStruct((N_GRAPHS, FDIM), jnp.float32),
        scratch_types=[
            pltpu.VMEM((CHUNK, FDIM), jnp.float32),
            pltpu.VMEM((CHUNK, FDIM), jnp.float32),
            pltpu.VMEM((N_GRAPHS,), jnp.int32),
            pltpu.VMEM((1, FDIM), jnp.float32),
            pltpu.SemaphoreType.DMA,
            pltpu.SemaphoreType.DMA,
        ],
    )(_pool_body)
    return f(feat2d, n_nodes)


def kernel(features, n_nodes):
    n = features.shape[0]
    feat2d = features.reshape(n, FDIM)
    means = _pool(feat2d, n_nodes)
    return means.reshape(N_GRAPHS, 4, 128).transpose(0, 2, 1).reshape(
        N_GRAPHS, FDIM)


# speedup vs baseline: 10.5842x; 10.5842x over previous
import functools

import jax
import jax.numpy as jnp
from jax import lax
from jax.experimental import pallas as pl
from jax.experimental.pallas import tpu as pltpu
from jax.experimental.pallas import tpu_sc as plsc

N_GRAPHS = 100
ROWS_PER_GRAPH = 1000
FDIM = 512
LANES = 16
VECS = FDIM // LANES
CHUNK = 125
N_CHUNKS = ROWS_PER_GRAPH // CHUNK
NW = 32
GRAPHS_PER_W = 4
NN_PAD = 128


def _pool_body(feat_hbm, nn_hbm, out_hbm, buf0, buf1, nn_v, acc,
               sem0, sem1):
    wid = lax.axis_index("c") * 16 + lax.axis_index("s")
    pltpu.sync_copy(nn_hbm, nn_v)
    bufs = (buf0, buf1)
    sems = (sem0, sem1)
    zero = jnp.zeros((LANES,), jnp.float32)

    for k in range(GRAPHS_PER_W):
        g = k * NW + wid

        @pl.when(g < N_GRAPHS)
        def _():
            base = g * (ROWS_PER_GRAPH * FDIM)
            pltpu.async_copy(
                feat_hbm.at[pl.ds(base, CHUNK * FDIM)], bufs[0],
                sems[0]).wait()
            for j in range(VECS):
                acc[pl.ds(j * LANES, LANES)] = zero
            for i in range(N_CHUNKS):
                cur = bufs[i % 2]
                if i + 1 < N_CHUNKS:
                    nxt = pltpu.async_copy(
                        feat_hbm.at[pl.ds(base + (i + 1) * (CHUNK * FDIM),
                                          CHUNK * FDIM)],
                        bufs[(i + 1) % 2], sems[(i + 1) % 2])

                def row_body(r, _):
                    rb = r * FDIM
                    for j in range(VECS):
                        plsc.addupdate(acc.at[pl.ds(j * LANES, LANES)],
                                       cur[pl.ds(rb + j * LANES, LANES)])
                    return 0

                lax.fori_loop(0, CHUNK, row_body, 0)
                if i + 1 < N_CHUNKS:
                    nxt.wait()

            n_f = jnp.maximum(nn_v[pl.ds(g, LANES)][0].astype(jnp.float32),
                              1.0)
            scale = 1.0 / jnp.full((LANES,), n_f, jnp.float32)
            for j in range(VECS):
                sl = pl.ds(j * LANES, LANES)
                acc[sl] = acc[sl] * scale
            pltpu.sync_copy(acc, out_hbm.at[pl.ds(g * FDIM, FDIM)])


@jax.jit
def _pool(feat_flat, nn_pad):
    mesh = plsc.VectorSubcoreMesh(core_axis_name="c", subcore_axis_name="s")
    f = functools.partial(
        pl.kernel,
        mesh=mesh,
        out_type=jax.ShapeDtypeStruct((N_GRAPHS * FDIM,), jnp.float32),
        scratch_types=[
            pltpu.VMEM((CHUNK * FDIM,), jnp.float32),
            pltpu.VMEM((CHUNK * FDIM,), jnp.float32),
            pltpu.VMEM((NN_PAD,), jnp.int32),
            pltpu.VMEM((FDIM,), jnp.float32),
            pltpu.SemaphoreType.DMA,
            pltpu.SemaphoreType.DMA,
        ],
    )(_pool_body)
    return f(feat_flat, nn_pad)


def kernel(features, n_nodes):
    feat_flat = features.reshape(-1)
    nn_pad = jnp.zeros((NN_PAD,), jnp.int32).at[:N_GRAPHS].set(n_nodes)
    means = _pool(feat_flat, nn_pad)
    return means.reshape(N_GRAPHS, 4, 128).transpose(0, 2, 1).reshape(
        N_GRAPHS, FDIM)


# speedup vs baseline: 36.9508x; 3.4911x over previous
import functools

import jax
import jax.numpy as jnp
from jax import lax
from jax.experimental import pallas as pl
from jax.experimental.pallas import tpu as pltpu
from jax.experimental.pallas import tpu_sc as plsc

N_GRAPHS = 100
ROWS_PER_GRAPH = 1000
FDIM = 512
LANES = 16
VECS = FDIM // LANES
CHUNK = 50
N_CHUNKS = ROWS_PER_GRAPH // CHUNK
NW = 32
GRAPHS_PER_W = 4
NN_PAD = 128


def _pool_body(feat_hbm, nn_hbm, out_hbm, buf0, buf1, nn_v, acc,
               sem0, sem1):
    wid = lax.axis_index("c") * 16 + lax.axis_index("s")
    pltpu.sync_copy(nn_hbm, nn_v)
    bufs = (buf0, buf1)
    sems = (sem0, sem1)
    zero = jnp.zeros((LANES,), jnp.float32)

    for k in range(GRAPHS_PER_W):
        g = k * NW + wid

        @pl.when(g < N_GRAPHS)
        def _():
            base = g * (ROWS_PER_GRAPH * FDIM)
            for j in range(VECS):
                acc[pl.ds(j * LANES, LANES)] = zero
            for b in range(2):
                pltpu.async_copy(
                    feat_hbm.at[pl.ds(base + b * (CHUNK * FDIM),
                                      CHUNK * FDIM)], bufs[b], sems[b])

            def chunk_pair(t, _):
                for b in range(2):
                    i = 2 * t + b
                    cur = bufs[b]
                    pltpu.make_async_copy(
                        feat_hbm.at[pl.ds(0, CHUNK * FDIM)], cur,
                        sems[b]).wait()

                    def row_body(r, accs):
                        rb = r * FDIM
                        return tuple(
                            accs[j] + cur[pl.ds(rb + j * LANES, LANES)]
                            for j in range(VECS))

                    accs = lax.fori_loop(0, CHUNK, row_body, (zero,) * VECS)

                    @pl.when(i + 2 < N_CHUNKS)
                    def _():
                        pltpu.async_copy(
                            feat_hbm.at[pl.ds(base + (i + 2) * (CHUNK * FDIM),
                                              CHUNK * FDIM)],
                            cur, sems[b])

                    for j in range(VECS):
                        plsc.addupdate(acc.at[pl.ds(j * LANES, LANES)],
                                       accs[j])
                return 0

            lax.fori_loop(0, N_CHUNKS // 2, chunk_pair, 0)

            n_f = jnp.maximum(nn_v[pl.ds(g, LANES)][0].astype(jnp.float32),
                              1.0)
            scale = 1.0 / jnp.full((LANES,), n_f, jnp.float32)
            for j in range(VECS):
                sl = pl.ds(j * LANES, LANES)
                acc[sl] = acc[sl] * scale
            pltpu.sync_copy(acc, out_hbm.at[pl.ds(g * FDIM, FDIM)])


@jax.jit
def _pool(feat_flat, nn_pad):
    mesh = plsc.VectorSubcoreMesh(core_axis_name="c", subcore_axis_name="s")
    f = functools.partial(
        pl.kernel,
        mesh=mesh,
        out_type=jax.ShapeDtypeStruct((N_GRAPHS * FDIM,), jnp.float32),
        scratch_types=[
            pltpu.VMEM((CHUNK * FDIM,), jnp.float32),
            pltpu.VMEM((CHUNK * FDIM,), jnp.float32),
            pltpu.VMEM((NN_PAD,), jnp.int32),
            pltpu.VMEM((FDIM,), jnp.float32),
            pltpu.SemaphoreType.DMA,
            pltpu.SemaphoreType.DMA,
        ],
    )(_pool_body)
    return f(feat_flat, nn_pad)


def kernel(features, n_nodes):
    feat_flat = features.reshape(-1)
    nn_pad = jnp.zeros((NN_PAD,), jnp.int32).at[:N_GRAPHS].set(n_nodes)
    means = _pool(feat_flat, nn_pad)
    return means.reshape(N_GRAPHS, 4, 128).transpose(0, 2, 1).reshape(
        N_GRAPHS, FDIM)


# speedup vs baseline: 41.6516x; 1.1272x over previous
import functools

import jax
import jax.numpy as jnp
from jax import lax
from jax.experimental import pallas as pl
from jax.experimental.pallas import tpu as pltpu
from jax.experimental.pallas import tpu_sc as plsc

N_GRAPHS = 100
ROWS_PER_GRAPH = 1000
FDIM = 512
LANES = 16
VECS = FDIM // LANES
CHUNK = 50
N_CHUNKS = ROWS_PER_GRAPH // CHUNK
NW = 32
FULL_GRAPHS_PER_W = 3
TAIL_ROWS = 125
TCHUNK = 25
N_TCHUNKS = TAIL_ROWS // TCHUNK
NN_PAD = 128
DUMMY_ROW = 15


def _pool_body(feat_hbm, nn_hbm, out_hbm, buf0, buf1, nn_v, acc,
               tail_sums, spacc, sem0, sem1):
    c = lax.axis_index("c")
    s = lax.axis_index("s")
    wid = c * 16 + s
    pltpu.sync_copy(nn_hbm, nn_v)
    bufs = (buf0, buf1)
    sems = (sem0, sem1)
    zero = jnp.zeros((LANES,), jnp.float32)

    for k in range(FULL_GRAPHS_PER_W):
        g = k * NW + wid
        base = g * (ROWS_PER_GRAPH * FDIM)
        for b in range(2):
            pltpu.async_copy(
                feat_hbm.at[pl.ds(base + b * (CHUNK * FDIM),
                                  CHUNK * FDIM)], bufs[b], sems[b])
        for j in range(VECS):
            acc[pl.ds(j * LANES, LANES)] = zero

        def chunk_pair(t, _):
            for b in range(2):
                i = 2 * t + b
                cur = bufs[b]
                pltpu.make_async_copy(
                    feat_hbm.at[pl.ds(0, CHUNK * FDIM)], cur,
                    sems[b]).wait()

                def row_body(r, accs):
                    rb = 2 * r * FDIM
                    accs = tuple(
                        accs[j] + cur[pl.ds(rb + j * LANES, LANES)]
                        for j in range(VECS))
                    return tuple(
                        accs[j] + cur[pl.ds(rb + FDIM + j * LANES, LANES)]
                        for j in range(VECS))

                accs = lax.fori_loop(0, CHUNK // 2, row_body,
                                     (zero,) * VECS)

                @pl.when(i + 2 < N_CHUNKS)
                def _():
                    pltpu.async_copy(
                        feat_hbm.at[pl.ds(base + (i + 2) * (CHUNK * FDIM),
                                          CHUNK * FDIM)],
                        cur, sems[b])

                for j in range(VECS):
                    plsc.addupdate(acc.at[pl.ds(j * LANES, LANES)],
                                   accs[j])
            return 0

        lax.fori_loop(0, N_CHUNKS // 2, chunk_pair, 0)

        n_f = jnp.maximum(nn_v[pl.ds(g, LANES)][0].astype(jnp.float32),
                          1.0)
        scale = 1.0 / jnp.full((LANES,), n_f, jnp.float32)
        for j in range(VECS):
            sl = pl.ds(j * LANES, LANES)
            acc[sl] = acc[sl] * scale
        pltpu.sync_copy(acc, out_hbm.at[pl.ds(g * FDIM, FDIM)])

    row_local = s // 8
    gt = 96 + 2 * c + row_local
    ut = s % 8
    tbase = (gt * ROWS_PER_GRAPH + ut * TAIL_ROWS) * FDIM
    for b in range(2):
        pltpu.async_copy(
            feat_hbm.at[pl.ds(tbase + b * (TCHUNK * FDIM), TCHUNK * FDIM)],
            bufs[b].at[pl.ds(0, TCHUNK * FDIM)], sems[b])
    taccs = (zero,) * VECS
    for i in range(N_TCHUNKS):
        cur = bufs[i % 2]
        pltpu.make_async_copy(
            feat_hbm.at[pl.ds(0, TCHUNK * FDIM)],
            cur.at[pl.ds(0, TCHUNK * FDIM)], sems[i % 2]).wait()

        def trow_body(r, accs):
            rb = r * FDIM
            return tuple(
                accs[j] + cur[pl.ds(rb + j * LANES, LANES)]
                for j in range(VECS))

        taccs = lax.fori_loop(0, TCHUNK, trow_body, taccs)
        if i + 2 < N_TCHUNKS:
            pltpu.async_copy(
                feat_hbm.at[pl.ds(tbase + (i + 2) * (TCHUNK * FDIM),
                                  TCHUNK * FDIM)],
                cur.at[pl.ds(0, TCHUNK * FDIM)], sems[i % 2])

    for j in range(VECS):
        acc[pl.ds(j * LANES, LANES)] = taccs[j]
    pltpu.sync_copy(acc, spacc.at[pl.ds(s * FDIM, FDIM)])
    plsc.subcore_barrier()

    @pl.when(s == 0)
    def _():
        pltpu.sync_copy(spacc, tail_sums)
        for half in range(2):
            gf = 96 + 2 * c + half
            n_f = jnp.maximum(
                nn_v[pl.ds(gf, LANES)][0].astype(jnp.float32), 1.0)
            scale = 1.0 / jnp.full((LANES,), n_f, jnp.float32)
            for j in range(VECS):
                v = tail_sums[pl.ds(8 * half * FDIM + j * LANES, LANES)]
                for r in range(1, 8):
                    v = v + tail_sums[pl.ds((8 * half + r) * FDIM
                                            + j * LANES, LANES)]
                acc[pl.ds(j * LANES, LANES)] = v * scale
            pltpu.sync_copy(acc, out_hbm.at[pl.ds(gf * FDIM, FDIM)])


@jax.jit
def _pool(feat_flat, nn_pad):
    mesh = plsc.VectorSubcoreMesh(core_axis_name="c", subcore_axis_name="s")
    f = functools.partial(
        pl.kernel,
        mesh=mesh,
        out_type=jax.ShapeDtypeStruct((N_GRAPHS * FDIM,), jnp.float32),
        scratch_types=[
            pltpu.VMEM((CHUNK * FDIM,), jnp.float32),
            pltpu.VMEM((CHUNK * FDIM,), jnp.float32),
            pltpu.VMEM((NN_PAD,), jnp.int32),
            pltpu.VMEM((FDIM,), jnp.float32),
            pltpu.VMEM((16 * FDIM,), jnp.float32),
            pltpu.VMEM_SHARED((16 * FDIM,), jnp.float32),
            pltpu.SemaphoreType.DMA,
            pltpu.SemaphoreType.DMA,
        ],
    )(_pool_body)
    return f(feat_flat, nn_pad)


def kernel(features, n_nodes):
    feat_flat = features.reshape(-1)
    nn_pad = jnp.zeros((NN_PAD,), jnp.int32).at[:N_GRAPHS].set(n_nodes)
    means = _pool(feat_flat, nn_pad)
    return means.reshape(N_GRAPHS, 4, 128).transpose(0, 2, 1).reshape(
        N_GRAPHS, FDIM)
